# baseline (speedup 1.0000x reference)
import jax
import jax.numpy as jnp
import numpy as np
from jax.experimental import pallas as pl
from jax.experimental.pallas import tpu as pltpu

_N = 10000
_E = 320000
_NWAVE = 8
_NANG = 9
_NORB = 32
_CUTOFF = 4.0
_INDEX_L = np.array([0, 1, 1, 1, 2, 2, 2, 2, 2])


def _apply_mlp(params, x):
    h = x
    for W, b in params[:-1]:
        h = h @ W + b
        mu = jnp.mean(h, axis=-1, keepdims=True)
        var = jnp.var(h, axis=-1, keepdims=True)
        h = (h - mu) / jnp.sqrt(var + 1e-5)
        h = jax.nn.silu(h)
    W, b = params[-1]
    return h @ W + b


def _sph_cal(v):
    x, y, z = v[0], v[1], v[2]
    r2 = x * x + y * y + z * z
    c0 = 0.28209479177387814
    c1 = 0.4886025119029199
    c2a = 1.0925484305920792
    c2b = 0.31539156525252005
    c2c = 0.5462742152960396
    return jnp.stack([
        c0 * jnp.ones_like(x),
        c1 * y, c1 * z, c1 * x,
        c2a * x * y, c2a * y * z, c2b * (3.0 * z * z - r2), c2a * x * z,
        c2c * (x * x - y * y)
    ], axis=0)


def _cutoff_cosine(d):
    t = 0.5 * jnp.cos(d * (np.pi / _CUTOFF)) + 0.5
    return t * t


def _final_sum_kernel(out_ref, cf_ref, acc_ref):
    acc_ref[...] = jnp.sum(out_ref[...] * cf_ref[...]).reshape(1, 1)


def kernel(cart, neighlist, shifts, center_factor, neigh_factor, species, params):
    idx_c = neighlist[0]
    idx_n = neighlist[1]
    distvec = cart[idx_n] - cart[idx_c] + shifts
    distances = jnp.linalg.norm(distvec, axis=1)
    center_coeff = _apply_mlp(params["embnn"], species)
    f0 = center_coeff[idx_c]
    f1 = center_coeff[idx_n]
    neigh_emb = (f0 * f1).T
    cut_distances = neigh_factor * _cutoff_cosine(distances)
    contracted = params["contracted_coeff"][_INDEX_L]
    alpha = neigh_emb[_NWAVE:2 * _NWAVE]
    rs = neigh_emb[2 * _NWAVE:3 * _NWAVE]
    radial = jnp.exp(-jnp.square(alpha * (distances[None, :] - rs)))
    sph = _sph_cal(distvec.T / _CUTOFF)
    orbital = jnp.einsum('i,ji,ji,ki->ikj', cut_distances, radial,
                         neigh_emb[:_NWAVE], sph)
    center_orbital = jnp.zeros((cart.shape[0], _NANG, _NWAVE),
                               dtype=cart.dtype).at[idx_c].add(orbital)
    contracted_orbital = jnp.einsum('ikj,kjm->ikm', center_orbital, contracted)
    density = jnp.einsum('ikm,ikm->im', contracted_orbital, contracted_orbital)
    for mlp in params["iters"]:
        iter_coeff = _apply_mlp(mlp, density)
        weight_orbital = jnp.einsum('ij,ikj->ikj', iter_coeff[idx_n], orbital) \
            + jnp.einsum('ikj,i->ikj', center_orbital[idx_n], cut_distances)
        center_orbital = center_orbital.at[idx_c].add(weight_orbital)
        contracted_orbital = jnp.einsum('ikj,kjm->ikm', center_orbital, contracted)
        density = density + jnp.einsum('ikm,ikm->im', contracted_orbital,
                                       contracted_orbital)
    output = _apply_mlp(params["outnn"], density)
    total = pl.pallas_call(
        _final_sum_kernel,
        out_shape=jax.ShapeDtypeStruct((1, 1), jnp.float32),
    )(output.reshape(_N, 1), center_factor.reshape(_N, 1))
    return total[0, 0]


# speedup vs baseline: 40.3240x; 40.3240x over previous
import functools

import jax
import jax.numpy as jnp
import numpy as np
from jax import lax
from jax.experimental import pallas as pl
from jax.experimental.pallas import tpu as pltpu
from jax.experimental.pallas import tpu_sc as plsc

_N = 10000
_NP = 10240
_E = 320000
_NWAVE = 8
_NANG = 9
_NORB = 32
_CUTOFF = 4.0
_INDEX_L = np.array([0, 1, 1, 1, 2, 2, 2, 2, 2])

_C = 2000
_NCH = _E // _C
_NT = 18
_NQ = 16
_EPQ = _E // _NQ
_CPQ = _EPQ // _C
_G = 4
_LB = 1280
_ND = _NP // _LB
_NW = 32
_EPW = _E // _NW
_CPW = _EPW // _C

_F32 = jnp.float32
_I32 = jnp.int32


def _ln_silu(h):
    mu = jnp.mean(h, axis=0, keepdims=True)
    var = jnp.var(h, axis=0, keepdims=True)
    h = (h - mu) / jnp.sqrt(var + 1e-5)
    return h * jax.nn.sigmoid(h)


def _tc_embnn_body(sp_ref, w1_ref, b1_ref, w2_ref, b2_ref, w3_ref, b3_ref,
                   out_ref):
    sp = sp_ref[...]
    h = _ln_silu(w1_ref[...] * sp + b1_ref[...])
    h = _ln_silu(jnp.dot(w2_ref[...], h, preferred_element_type=_F32)
                 + b2_ref[...])
    out_ref[...] = (jnp.dot(w3_ref[...], h, preferred_element_type=_F32)
                    + b3_ref[...])


def _tc_embnn(spT, w1, b1, w2, b2, w3, b3):
    return pl.pallas_call(
        _tc_embnn_body,
        out_shape=jax.ShapeDtypeStruct((24, _NP), _F32),
    )(spT, w1, b1, w2, b2, w3, b3)


def _sc_edge_gather(table, ic, inn):
    mesh = plsc.VectorSubcoreMesh(core_axis_name="c", subcore_axis_name="s")

    @functools.partial(
        pl.kernel, mesh=mesh,
        compiler_params=pltpu.CompilerParams(needs_layout_passes=False),
        out_type=jax.ShapeDtypeStruct((32, _NCH, 1, _C), _F32),
        scratch_types=[
            pltpu.VMEM((_NP,), _F32),
            pltpu.VMEM((_C,), _I32),
            pltpu.VMEM((_C,), _I32),
            pltpu.VMEM((_C,), _F32),
        ],
    )
    def k(table_hbm, ic_hbm, in_hbm, ge_hbm, col_v, ic_v, in_v, out_v):
        wid = lax.axis_index("s") * 2 + lax.axis_index("c")

        def row_body(row, _):
            pltpu.sync_copy(table_hbm.at[0, pl.ds(row * _NP, _NP)], col_v)

            def ch_body(ch, _):
                base = wid * _EPW + ch * _C
                pltpu.sync_copy(ic_hbm.at[pl.ds(base, _C)], ic_v)
                pltpu.sync_copy(in_hbm.at[pl.ds(base, _C)], in_v)

                def g_body(g, _):
                    ids_c = ic_v[pl.ds(g * 16, 16)]
                    ids_n = in_v[pl.ds(g * 16, 16)]
                    gc = plsc.load_gather(col_v, [ids_c])
                    gn = plsc.load_gather(col_v, [ids_n])
                    out_v[pl.ds(g * 16, 16)] = jnp.where(
                        row < 3, gn - gc, gn * gc)
                    return 0

                lax.fori_loop(0, _C // 16, g_body, 0)
                pltpu.sync_copy(out_v, ge_hbm.at[row, wid * _CPW + ch, 0])
                return 0

            lax.fori_loop(0, _CPW, ch_body, 0)
            return 0

        lax.fori_loop(0, 27, row_body, 0)

    return k(table, ic, inn)


def _tc_edge_body(ge_ref, sh_ref, nf_ref, nl_ref, out_ref):
    ge = ge_ref[:, 0, 0, :]
    dv = ge[0:3] + sh_ref[0]
    r2 = jnp.sum(dv * dv, axis=0, keepdims=True)
    d = jnp.sqrt(r2)
    emb0 = ge[3:11]
    alpha = ge[11:19]
    rs = ge[19:27]
    t = 0.5 * jnp.cos(d * (np.pi / _CUTOFF)) + 0.5
    cut = nf_ref[0] * t * t
    arg = alpha * (d - rs)
    rad = jnp.exp(-(arg * arg))
    w = cut * rad * emb0
    s = dv * (1.0 / _CUTOFF)
    x, y, z = s[0:1], s[1:2], s[2:3]
    r2s = r2 * (1.0 / (_CUTOFF * _CUTOFF))
    c0 = 0.28209479177387814
    c1 = 0.4886025119029199
    c2a = 1.0925484305920792
    c2b = 0.31539156525252005
    c2c = 0.5462742152960396
    sph = jnp.concatenate([
        jnp.full((1, _C), c0, _F32),
        c1 * y, c1 * z, c1 * x,
        c2a * x * y, c2a * y * z, c2b * (3.0 * z * z - r2s), c2a * x * z,
        c2c * (x * x - y * y)], axis=0)
    icf = lax.bitcast_convert_type(nl_ref[0, 0:1], _F32)
    inf_ = lax.bitcast_convert_type(nl_ref[0, 1:2], _F32)
    segs = []
    for kk in range(_NANG):
        for jh in range(2):
            segs += [w[jh * 4:jh * 4 + 4], sph[kk:kk + 1], cut, icf, inf_]
    out_ref[0] = jnp.concatenate(segs, axis=0)


def _tc_edge_math(ge, shR, nfR, nlR):
    return pl.pallas_call(
        _tc_edge_body,
        grid=(_NCH,),
        in_specs=[
            pl.BlockSpec((32, 1, 1, _C), lambda i: (0, i, 0, 0)),
            pl.BlockSpec((1, 3, _C), lambda i: (i, 0, 0)),
            pl.BlockSpec((1, 1, _C), lambda i: (i, 0, 0)),
            pl.BlockSpec((1, 2, _C), lambda i: (i, 0, 0)),
        ],
        out_specs=pl.BlockSpec((1, 144, _C), lambda i: (i, 0, 0)),
        out_shape=jax.ShapeDtypeStruct((_NCH, 144, _C), _F32),
    )(ge, shR, nfR, nlR)


def _sc_scatter0(P):
    mesh = plsc.VectorSubcoreMesh(core_axis_name="c", subcore_axis_name="s")

    @functools.partial(
        pl.kernel, mesh=mesh,
        compiler_params=pltpu.CompilerParams(needs_layout_passes=False),
        out_type=jax.ShapeDtypeStruct((_NQ, _NT, _G, _NP), _F32),
        scratch_types=[
            pltpu.VMEM((_G, _NP), _F32),
            pltpu.VMEM((8, _C), _F32),
        ],
    )
    def k(p_hbm, out_hbm, acc_v, b_v):
        wid = lax.axis_index("s") * 2 + lax.axis_index("c")
        jh = wid // _NQ
        q = wid % _NQ

        def task_body(kk, _):
            t = kk * 2 + jh

            def z_body(z, _):
                acc_v[z % _G, pl.ds((z // _G) * 16, 16)] = jnp.zeros((16,), _F32)
                return 0

            lax.fori_loop(0, _G * (_NP // 16), z_body, 0)

            def ch_body(ch, _):
                chg = q * _CPQ + ch
                pltpu.sync_copy(p_hbm.at[chg, pl.ds(t * 8, 8)], b_v)

                def g_body(g, _):
                    sl = pl.ds(g * 16, 16)
                    ids_c = plsc.bitcast(b_v[6, sl], _I32)
                    sph = b_v[4, sl]
                    for gg in range(_G):
                        val = sph * b_v[gg, sl]
                        plsc.addupdate_scatter(
                            acc_v, [jnp.full((16,), gg, _I32), ids_c], val)
                    return 0

                lax.fori_loop(0, _C // 16, g_body, 0)
                return 0

            lax.fori_loop(0, _CPQ, ch_body, 0)
            pltpu.sync_copy(acc_v, out_hbm.at[q, t])
            return 0

        lax.fori_loop(0, 9, task_body, 0)

    return k(P)


def _sc_gather_q(Q, inn):
    mesh = plsc.VectorSubcoreMesh(core_axis_name="c", subcore_axis_name="s")

    @functools.partial(
        pl.kernel, mesh=mesh,
        compiler_params=pltpu.CompilerParams(needs_layout_passes=False),
        out_type=jax.ShapeDtypeStruct((_NCH, 8, _C), _F32),
        scratch_types=[
            pltpu.VMEM((8, _NP), _F32),
            pltpu.VMEM((_C,), _I32),
            pltpu.VMEM((8, _C), _F32),
        ],
    )
    def k(q_hbm, in_hbm, qe_hbm, tab_v, in_v, out_v):
        wid = lax.axis_index("s") * 2 + lax.axis_index("c")
        pltpu.sync_copy(q_hbm, tab_v)

        def ch_body(ch, _):
            base = wid * _EPW + ch * _C
            pltpu.sync_copy(in_hbm.at[pl.ds(base, _C)], in_v)

            def g_body(g, _):
                ids_n = in_v[pl.ds(g * 16, 16)]
                for r in range(8):
                    out_v[r, pl.ds(g * 16, 16)] = plsc.load_gather(
                        tab_v, [jnp.full((16,), r, _I32), ids_n])
                return 0

            lax.fori_loop(0, _C // 16, g_body, 0)
            pltpu.sync_copy(out_v, qe_hbm.at[wid * _CPW + ch])
            return 0

        lax.fori_loop(0, _CPW, ch_body, 0)

    return k(Q, inn)


def _sc_scatter_iter(P, QE, CO):
    mesh = plsc.VectorSubcoreMesh(core_axis_name="c", subcore_axis_name="s")

    @functools.partial(
        pl.kernel, mesh=mesh,
        compiler_params=pltpu.CompilerParams(needs_layout_passes=False),
        out_type=jax.ShapeDtypeStruct((_NQ, _NT, _G, _NP), _F32),
        scratch_types=[
            pltpu.VMEM((_G, _NP), _F32),
            pltpu.VMEM((_G, _NP), _F32),
            pltpu.VMEM((8, _C), _F32),
            pltpu.VMEM((8, _C), _F32),
        ],
    )
    def k(p_hbm, qe_hbm, co_hbm, out_hbm, cot_v, acc_v, b_v, q_v):
        wid = lax.axis_index("s") * 2 + lax.axis_index("c")
        jh = wid // _NQ
        q = wid % _NQ

        def task_body(kk, _):
            t = kk * 2 + jh
            pltpu.sync_copy(co_hbm.at[t], cot_v)

            def z_body(z, _):
                acc_v[z % _G, pl.ds((z // _G) * 16, 16)] = jnp.zeros((16,), _F32)
                return 0

            lax.fori_loop(0, _G * (_NP // 16), z_body, 0)

            def ch_body(ch, _):
                chg = q * _CPQ + ch
                pltpu.sync_copy(p_hbm.at[chg, pl.ds(t * 8, 8)], b_v)
                pltpu.sync_copy(qe_hbm.at[chg], q_v)

                def g_body(g, _):
                    sl = pl.ds(g * 16, 16)
                    cut = b_v[5, sl]
                    ids_c = plsc.bitcast(b_v[6, sl], _I32)
                    ids_n = plsc.bitcast(b_v[7, sl], _I32)
                    sph = b_v[4, sl]
                    for gg in range(_G):
                        gv = jnp.full((16,), gg, _I32)
                        con = plsc.load_gather(cot_v, [gv, ids_n])
                        val = (q_v[jh * 4 + gg, sl] * (sph * b_v[gg, sl])
                               + cut * con)
                        plsc.addupdate_scatter(acc_v, [gv, ids_c], val)
                    return 0

                lax.fori_loop(0, _C // 16, g_body, 0)
                return 0

            lax.fori_loop(0, _CPQ, ch_body, 0)
            pltpu.sync_copy(acc_v, out_hbm.at[q, t])
            return 0

        lax.fori_loop(0, 9, task_body, 0)

    return k(P, QE, CO)


def _density(co, ct_ref):
    dens = jnp.zeros((_NORB, _LB), _F32)
    for kk in range(_NANG):
        co8 = jnp.concatenate([co[2 * kk], co[2 * kk + 1]], axis=0)
        gk = jnp.dot(ct_ref[kk], co8, preferred_element_type=_F32)
        dens = dens + gk * gk
    return dens


def _mlp(d, w1_ref, b1_ref, w2_ref, b2_ref, w3_ref, b3_ref):
    h = _ln_silu(jnp.dot(w1_ref[...], d, preferred_element_type=_F32)
                 + b1_ref[...])
    h = _ln_silu(jnp.dot(w2_ref[...], h, preferred_element_type=_F32)
                 + b2_ref[...])
    return jnp.dot(w3_ref[...], h, preferred_element_type=_F32) + b3_ref[...]


def _tc_dense_a_body(parts_ref, ct_ref, w1, b1, w2, b2, w3, b3,
                     co_ref, d_ref, q_ref):
    co = jnp.sum(parts_ref[...], axis=0)
    dens = _density(co, ct_ref)
    co_ref[...] = co
    d_ref[...] = dens
    q_ref[...] = _mlp(dens, w1, b1, w2, b2, w3, b3)


def _tc_dense_b_body(parts_ref, cop_ref, dp_ref, ct_ref, w1, b1, w2, b2,
                     w3, b3, co_ref, d_ref, q_ref):
    co = cop_ref[...] + jnp.sum(parts_ref[...], axis=0)
    dens = dp_ref[...] + _density(co, ct_ref)
    co_ref[...] = co
    d_ref[...] = dens
    q_ref[...] = _mlp(dens, w1, b1, w2, b2, w3, b3)


def _tc_final_body(parts_ref, cop_ref, dp_ref, ct_ref, w1, b1, w2, b2,
                   w3, b3, cf_ref, out_ref):
    i = pl.program_id(0)
    co = cop_ref[...] + jnp.sum(parts_ref[...], axis=0)
    dens = dp_ref[...] + _density(co, ct_ref)
    o = _mlp(dens, w1, b1, w2, b2, w3, b3)
    part = jnp.sum(o * cf_ref[...]).reshape(1, 1)

    @pl.when(i == 0)
    def _():
        out_ref[...] = part

    @pl.when(i != 0)
    def _():
        out_ref[...] = out_ref[...] + part


def _dense_specs(extra_co):
    specs = [pl.BlockSpec((_NQ, _NT, _G, _LB), lambda i: (0, 0, 0, i))]
    if extra_co:
        specs += [pl.BlockSpec((_NT, _G, _LB), lambda i: (0, 0, i)),
                  pl.BlockSpec((32, _LB), lambda i: (0, i))]
    specs += [pl.BlockSpec((_NANG, _NORB, 8), lambda i: (0, 0, 0))]
    specs += [pl.BlockSpec(None, lambda i: (0, 0))] * 6
    return specs


_DENSE_OUT_SPECS = [
    pl.BlockSpec((_NT, _G, _LB), lambda i: (0, 0, i)),
    pl.BlockSpec((32, _LB), lambda i: (0, i)),
    pl.BlockSpec((8, _LB), lambda i: (0, i)),
]
_DENSE_OUT_SHAPE = [
    jax.ShapeDtypeStruct((_NT, _G, _NP), _F32),
    jax.ShapeDtypeStruct((32, _NP), _F32),
    jax.ShapeDtypeStruct((8, _NP), _F32),
]


def _tc_dense_a(parts, ct, ws):
    return pl.pallas_call(
        _tc_dense_a_body,
        grid=(_ND,),
        in_specs=_dense_specs(False),
        out_specs=_DENSE_OUT_SPECS,
        out_shape=_DENSE_OUT_SHAPE,
    )(parts, ct, *ws)


def _tc_dense_b(parts, co_p, d_p, ct, ws):
    return pl.pallas_call(
        _tc_dense_b_body,
        grid=(_ND,),
        in_specs=_dense_specs(True),
        out_specs=_DENSE_OUT_SPECS,
        out_shape=_DENSE_OUT_SHAPE,
    )(parts, co_p, d_p, ct, *ws)


def _tc_final(parts, co_p, d_p, ct, ws, cfT):
    specs = _dense_specs(True) + [pl.BlockSpec((1, _LB), lambda i: (0, i))]
    return pl.pallas_call(
        _tc_final_body,
        grid=(_ND,),
        in_specs=specs,
        out_specs=pl.BlockSpec((1, 1), lambda i: (0, 0)),
        out_shape=jax.ShapeDtypeStruct((1, 1), _F32),
    )(parts, co_p, d_p, ct, *ws, cfT)


def _prep_mlp(params):
    (w1, b1), (w2, b2), (w3, b3) = params
    return (w1.T, b1.reshape(-1, 1), w2.T, b2.reshape(-1, 1),
            w3.T, b3.reshape(-1, 1))


def kernel(cart, neighlist, shifts, center_factor, neigh_factor, species,
           params):
    nl = neighlist.astype(_I32)
    ic = nl[0]
    inn = nl[1]
    cartT = jnp.pad(cart.T, ((0, 0), (0, _NP - _N)))
    spT = jnp.pad(species.T, ((0, 0), (0, _NP - _N)))
    cfT = jnp.pad(center_factor.reshape(1, -1), ((0, 0), (0, _NP - _N)))
    shR = shifts.T.reshape(3, _NCH, _C).transpose(1, 0, 2)
    nfR = neigh_factor.reshape(1, _NCH, _C).transpose(1, 0, 2)
    nlR = nl.reshape(2, _NCH, _C).transpose(1, 0, 2)
    ct = jnp.transpose(params["contracted_coeff"][_INDEX_L], (0, 2, 1))
    emb_ws = _prep_mlp(params["embnn"])
    iter_ws = [_prep_mlp(p) for p in params["iters"]]
    out_ws = _prep_mlp(params["outnn"])

    coeff = _tc_embnn(spT, *emb_ws)
    table = jnp.concatenate(
        [cartT, coeff, jnp.zeros((5, _NP), _F32)], axis=0).reshape(1, -1)
    ge = _sc_edge_gather(table, ic, inn)
    P = _tc_edge_math(ge, shR, nfR, nlR)
    parts = _sc_scatter0(P)
    co, dens, qv = _tc_dense_a(parts, ct, iter_ws[0])
    for it in range(1, 3):
        qe = _sc_gather_q(qv, inn)
        parts = _sc_scatter_iter(P, qe, co)
        co, dens, qv = _tc_dense_b(parts, co, dens, ct, iter_ws[it])
    qe = _sc_gather_q(qv, inn)
    parts = _sc_scatter_iter(P, qe, co)
    total = _tc_final(parts, co, dens, ct, out_ws, cfT)
    return total[0, 0]


# speedup vs baseline: 48.5031x; 1.2028x over previous
import functools

import jax
import jax.numpy as jnp
import numpy as np
from jax import lax
from jax.experimental import pallas as pl
from jax.experimental.pallas import tpu as pltpu
from jax.experimental.pallas import tpu_sc as plsc

_N = 10000
_NP = 10112
_E = 320000
_NWAVE = 8
_NANG = 9
_NORB = 32
_CUTOFF = 4.0
_INDEX_L = np.array([0, 1, 1, 1, 2, 2, 2, 2, 2])

_C = 400
_NCH = 800
_C2 = 2000
_NCH2 = 160
_NT = 18
_NQ = 16
_EPQ = _E // _NQ
_CPQ = _EPQ // _C
_G = 4
_LB = 128
_ND = _NP // _LB
_NW = 32
_EPW = _E // _NW
_CPW = _EPW // _C

_F32 = jnp.float32
_I32 = jnp.int32
_SCP = pltpu.CompilerParams(needs_layout_passes=False)


def _ln_silu(h):
    mu = jnp.mean(h, axis=0, keepdims=True)
    var = jnp.var(h, axis=0, keepdims=True)
    h = (h - mu) / jnp.sqrt(var + 1e-5)
    return h * jax.nn.sigmoid(h)


def _tc_embnn_body(sp_ref, w1_ref, b1_ref, w2_ref, b2_ref, w3_ref, b3_ref,
                   out_ref):
    sp = sp_ref[...]
    h = _ln_silu(w1_ref[...] * sp + b1_ref[...])
    h = _ln_silu(jnp.dot(w2_ref[...], h, preferred_element_type=_F32)
                 + b2_ref[...])
    out_ref[...] = (jnp.dot(w3_ref[...], h, preferred_element_type=_F32)
                    + b3_ref[...])


def _tc_embnn(spT, w1, b1, w2, b2, w3, b3):
    return pl.pallas_call(
        _tc_embnn_body,
        out_shape=jax.ShapeDtypeStruct((24, _NP), _F32),
    )(spT, w1, b1, w2, b2, w3, b3)


def _sc_edge_gather(table, ic, inn):
    mesh = plsc.VectorSubcoreMesh(core_axis_name="c", subcore_axis_name="s")

    @functools.partial(
        pl.kernel, mesh=mesh, compiler_params=_SCP,
        out_type=jax.ShapeDtypeStruct((32, _NCH, 1, _C), _F32),
        scratch_types=[
            pltpu.VMEM((2, _NP), _F32),
            pltpu.VMEM((_EPW,), _I32),
            pltpu.VMEM((_EPW,), _I32),
            pltpu.VMEM((2, _CPW, 1, _C), _F32),
            pltpu.SemaphoreType.DMA,
            pltpu.SemaphoreType.DMA,
            pltpu.SemaphoreType.DMA,
            pltpu.SemaphoreType.DMA,
            pltpu.SemaphoreType.DMA,
        ],
    )
    def k(table_hbm, ic_hbm, in_hbm, ge_hbm, col_v, ic_v, in_v, out_v,
          semi, semc0, semc1, semo0, semo1):
        wid = lax.axis_index("s") * 2 + lax.axis_index("c")
        ebase = wid * _EPW
        semc = (semc0, semc1)
        semo = (semo0, semo1)
        pltpu.async_copy(ic_hbm.at[pl.ds(ebase, _EPW)], ic_v, semi)
        pltpu.async_copy(in_hbm.at[pl.ds(ebase, _EPW)], in_v, semi)
        pltpu.async_copy(table_hbm.at[0, pl.ds(0, _NP)], col_v.at[0], semc0)
        pltpu.make_async_copy(ic_hbm.at[pl.ds(ebase, _EPW)], ic_v, semi).wait()
        pltpu.make_async_copy(in_hbm.at[pl.ds(ebase, _EPW)], in_v, semi).wait()

        for row in range(27):
            s = row % 2
            if row + 1 < 27:
                pltpu.async_copy(table_hbm.at[0, pl.ds((row + 1) * _NP, _NP)],
                                 col_v.at[(row + 1) % 2], semc[(row + 1) % 2])
            pltpu.make_async_copy(table_hbm.at[0, pl.ds(row * _NP, _NP)],
                                  col_v.at[s], semc[s]).wait()
            if row >= 2:
                pltpu.make_async_copy(
                    out_v.at[s], ge_hbm.at[row, pl.ds(wid * _CPW, _CPW)],
                    semo[s]).wait()

            def ch_body(ch, _):
                def g_body(g5, _):
                    for u in range(5):
                        g = g5 * 5 + u
                        off = ch * _C + g * 16
                        ids_c = ic_v[pl.ds(off, 16)]
                        ids_n = in_v[pl.ds(off, 16)]
                        gc = plsc.load_gather(
                            col_v, [jnp.full((16,), s, _I32), ids_c])
                        gn = plsc.load_gather(
                            col_v, [jnp.full((16,), s, _I32), ids_n])
                        v = (gn - gc) if row < 3 else (gn * gc)
                        out_v[s, ch, 0, pl.ds(g * 16, 16)] = v
                    return 0

                lax.fori_loop(0, _C // 80, g_body, 0)
                return 0

            lax.fori_loop(0, _CPW, ch_body, 0)
            pltpu.async_copy(out_v.at[s],
                             ge_hbm.at[row, pl.ds(wid * _CPW, _CPW)], semo[s])
        pltpu.make_async_copy(
            out_v.at[0], ge_hbm.at[26, pl.ds(wid * _CPW, _CPW)], semo[0]).wait()
        pltpu.make_async_copy(
            out_v.at[1], ge_hbm.at[25, pl.ds(wid * _CPW, _CPW)], semo[1]).wait()

    return k(table, ic, inn)


def _tc_edge_body(ge_ref, sh_ref, nf_ref, nl_ref, out_ref):
    ge = ge_ref[:, 0, 0, :]
    dv = ge[0:3] + sh_ref[0]
    r2 = jnp.sum(dv * dv, axis=0, keepdims=True)
    d = jnp.sqrt(r2)
    emb0 = ge[3:11]
    alpha = ge[11:19]
    rs = ge[19:27]
    t = 0.5 * jnp.cos(d * (np.pi / _CUTOFF)) + 0.5
    cut = nf_ref[0] * t * t
    arg = alpha * (d - rs)
    rad = jnp.exp(-(arg * arg))
    w = cut * rad * emb0
    s = dv * (1.0 / _CUTOFF)
    x, y, z = s[0:1], s[1:2], s[2:3]
    r2s = r2 * (1.0 / (_CUTOFF * _CUTOFF))
    c0 = 0.28209479177387814
    c1 = 0.4886025119029199
    c2a = 1.0925484305920792
    c2b = 0.31539156525252005
    c2c = 0.5462742152960396
    sph = jnp.concatenate([
        jnp.full((1, _C), c0, _F32),
        c1 * y, c1 * z, c1 * x,
        c2a * x * y, c2a * y * z, c2b * (3.0 * z * z - r2s), c2a * x * z,
        c2c * (x * x - y * y)], axis=0)
    icf = lax.bitcast_convert_type(nl_ref[0, 0:1], _F32)
    inf_ = lax.bitcast_convert_type(nl_ref[0, 1:2], _F32)
    segs = []
    for kk in range(_NANG):
        for jh in range(2):
            segs += [w[jh * 4:jh * 4 + 4], sph[kk:kk + 1], cut, icf, inf_]
    out_ref[0] = jnp.concatenate(segs, axis=0)


def _tc_edge_math(ge, shR, nfR, nlR):
    return pl.pallas_call(
        _tc_edge_body,
        grid=(_NCH,),
        in_specs=[
            pl.BlockSpec((32, 1, 1, _C), lambda i: (0, i, 0, 0)),
            pl.BlockSpec((1, 3, _C), lambda i: (i, 0, 0)),
            pl.BlockSpec((1, 1, _C), lambda i: (i, 0, 0)),
            pl.BlockSpec((1, 2, _C), lambda i: (i, 0, 0)),
        ],
        out_specs=pl.BlockSpec((1, 144, _C), lambda i: (i, 0, 0)),
        out_shape=jax.ShapeDtypeStruct((_NCH, 144, _C), _F32),
    )(ge, shR, nfR, nlR)


def _zero_acc(acc_v):
    def z_body(z8, _):
        zv = jnp.zeros((16,), _F32)
        for u in range(8):
            z = z8 * 8 + u
            for gg in range(_G):
                acc_v[gg, pl.ds(z * 16, 16)] = zv
        return 0

    lax.fori_loop(0, _NP // 128, z_body, 0)


def _scatter_kernel(iter_mode):
    mesh = plsc.VectorSubcoreMesh(core_axis_name="c", subcore_axis_name="s")
    scratch = [
        pltpu.VMEM((_G, _NP), _F32),
        pltpu.VMEM((2, 8, _C), _F32),
        pltpu.SemaphoreType.DMA,
        pltpu.SemaphoreType.DMA,
        pltpu.SemaphoreType.DMA,
    ]
    if iter_mode:
        scratch = [
            pltpu.VMEM((_G, _NP), _F32),
            pltpu.VMEM((_G, _NP), _F32),
            pltpu.SemaphoreType.DMA,
        ] + scratch

    def body(*args):
        if iter_mode:
            (p_hbm, q_hbm, co_hbm, out_hbm, cot_v, qt_v, semt,
             acc_v, b_v, semb0, semb1, semw) = args
        else:
            (p_hbm, out_hbm, acc_v, b_v, semb0, semb1, semw) = args
        wid = lax.axis_index("s") * 2 + lax.axis_index("c")
        jh = wid // _NQ
        q = wid % _NQ
        semb = (semb0, semb1)
        if iter_mode:
            pltpu.async_copy(q_hbm.at[jh], qt_v, semt)

        for kk in range(9):
            t = kk * 2 + jh
            if iter_mode:
                pltpu.async_copy(co_hbm.at[t], cot_v, semt)
            if kk > 0:
                pltpu.make_async_copy(acc_v, out_hbm.at[q, t], semw).wait()
            _zero_acc(acc_v)
            if iter_mode:
                pltpu.make_async_copy(co_hbm.at[t], cot_v, semt).wait()
                if kk == 0:
                    pltpu.make_async_copy(q_hbm.at[jh], qt_v, semt).wait()
            cbase = q * _CPQ
            pltpu.async_copy(p_hbm.at[cbase, pl.ds(t * 8, 8)], b_v.at[0],
                             semb0)

            def compute(s, chg):
                def g_body(g5, _):
                    for u in range(5):
                        g = g5 * 5 + u
                        sl = pl.ds(g * 16, 16)
                        ids_c = plsc.bitcast(b_v[s, 6, sl], _I32)
                        sph = b_v[s, 4, sl]
                        if iter_mode:
                            cut = b_v[s, 5, sl]
                            ids_n = plsc.bitcast(b_v[s, 7, sl], _I32)
                        for gg in range(_G):
                            gv = jnp.full((16,), gg, _I32)
                            val = sph * b_v[s, gg, sl]
                            if iter_mode:
                                con = plsc.load_gather(cot_v, [gv, ids_n])
                                qn = plsc.load_gather(qt_v, [gv, ids_n])
                                val = qn * val + cut * con
                            plsc.addupdate_scatter(acc_v, [gv, ids_c], val)
                    return 0

                lax.fori_loop(0, _C // 80, g_body, 0)

            def pair_body(p, _):
                chg0 = cbase + 2 * p
                pltpu.async_copy(p_hbm.at[chg0 + 1, pl.ds(t * 8, 8)],
                                 b_v.at[1], semb1)
                pltpu.make_async_copy(p_hbm.at[chg0, pl.ds(t * 8, 8)],
                                      b_v.at[0], semb0).wait()
                compute(0, chg0)

                @pl.when(p < _CPQ // 2 - 1)
                def _():
                    pltpu.async_copy(p_hbm.at[chg0 + 2, pl.ds(t * 8, 8)],
                                     b_v.at[0], semb0)

                pltpu.make_async_copy(p_hbm.at[chg0 + 1, pl.ds(t * 8, 8)],
                                      b_v.at[1], semb1).wait()
                compute(1, chg0 + 1)
                return 0

            lax.fori_loop(0, _CPQ // 2, pair_body, 0)
            pltpu.async_copy(acc_v, out_hbm.at[q, t], semw)
        pltpu.make_async_copy(acc_v, out_hbm.at[q, 16 + jh], semw).wait()

    return mesh, scratch, body


def _sc_scatter0(P):
    mesh, scratch, body = _scatter_kernel(False)
    k = pl.kernel(body, mesh=mesh, compiler_params=_SCP,
                  out_type=jax.ShapeDtypeStruct((_NQ, _NT, _G, _NP), _F32),
                  scratch_types=scratch)
    return k(P)


def _sc_scatter_iter(P, Q, CO):
    mesh, scratch, body = _scatter_kernel(True)
    k = pl.kernel(body, mesh=mesh, compiler_params=_SCP,
                  out_type=jax.ShapeDtypeStruct((_NQ, _NT, _G, _NP), _F32),
                  scratch_types=scratch)
    return k(P, Q, CO)


def _density(co, ct_ref):
    dens = jnp.zeros((_NORB, _LB), _F32)
    for kk in range(_NANG):
        co8 = jnp.concatenate([co[2 * kk], co[2 * kk + 1]], axis=0)
        gk = jnp.dot(ct_ref[kk], co8, preferred_element_type=_F32)
        dens = dens + gk * gk
    return dens


def _mlp(d, w1_ref, b1_ref, w2_ref, b2_ref, w3_ref, b3_ref):
    h = _ln_silu(jnp.dot(w1_ref[...], d, preferred_element_type=_F32)
                 + b1_ref[...])
    h = _ln_silu(jnp.dot(w2_ref[...], h, preferred_element_type=_F32)
                 + b2_ref[...])
    return jnp.dot(w3_ref[...], h, preferred_element_type=_F32) + b3_ref[...]


def _tc_dense_a_body(parts_ref, ct_ref, w1, b1, w2, b2, w3, b3,
                     co_ref, d_ref, q_ref):
    co = jnp.sum(parts_ref[...], axis=0)
    dens = _density(co, ct_ref)
    co_ref[...] = co
    d_ref[...] = dens
    q_ref[...] = _mlp(dens, w1, b1, w2, b2, w3, b3).reshape(2, 4, _LB)


def _tc_dense_b_body(parts_ref, cop_ref, dp_ref, ct_ref, w1, b1, w2, b2,
                     w3, b3, co_ref, d_ref, q_ref):
    co = cop_ref[...] + jnp.sum(parts_ref[...], axis=0)
    dens = dp_ref[...] + _density(co, ct_ref)
    co_ref[...] = co
    d_ref[...] = dens
    q_ref[...] = _mlp(dens, w1, b1, w2, b2, w3, b3).reshape(2, 4, _LB)


def _tc_final_body(parts_ref, cop_ref, dp_ref, ct_ref, w1, b1, w2, b2,
                   w3, b3, cf_ref, out_ref):
    i = pl.program_id(0)
    co = cop_ref[...] + jnp.sum(parts_ref[...], axis=0)
    dens = dp_ref[...] + _density(co, ct_ref)
    o = _mlp(dens, w1, b1, w2, b2, w3, b3)
    part = jnp.sum(o * cf_ref[...]).reshape(1, 1)

    @pl.when(i == 0)
    def _():
        out_ref[...] = part

    @pl.when(i != 0)
    def _():
        out_ref[...] = out_ref[...] + part


def _dense_specs(extra_co):
    specs = [pl.BlockSpec((_NQ, _NT, _G, _LB), lambda i: (0, 0, 0, i))]
    if extra_co:
        specs += [pl.BlockSpec((_NT, _G, _LB), lambda i: (0, 0, i)),
                  pl.BlockSpec((32, _LB), lambda i: (0, i))]
    specs += [pl.BlockSpec((_NANG, _NORB, 8), lambda i: (0, 0, 0))]
    specs += [pl.BlockSpec(None, lambda i: (0, 0))] * 6
    return specs


_DENSE_OUT_SPECS = [
    pl.BlockSpec((_NT, _G, _LB), lambda i: (0, 0, i)),
    pl.BlockSpec((32, _LB), lambda i: (0, i)),
    pl.BlockSpec((2, 4, _LB), lambda i: (0, 0, i)),
]
_DENSE_OUT_SHAPE = [
    jax.ShapeDtypeStruct((_NT, _G, _NP), _F32),
    jax.ShapeDtypeStruct((32, _NP), _F32),
    jax.ShapeDtypeStruct((2, 4, _NP), _F32),
]


def _tc_dense_a(parts, ct, ws):
    return pl.pallas_call(
        _tc_dense_a_body,
        grid=(_ND,),
        in_specs=_dense_specs(False),
        out_specs=_DENSE_OUT_SPECS,
        out_shape=_DENSE_OUT_SHAPE,
    )(parts, ct, *ws)


def _tc_dense_b(parts, co_p, d_p, ct, ws):
    return pl.pallas_call(
        _tc_dense_b_body,
        grid=(_ND,),
        in_specs=_dense_specs(True),
        out_specs=_DENSE_OUT_SPECS,
        out_shape=_DENSE_OUT_SHAPE,
    )(parts, co_p, d_p, ct, *ws)


def _tc_final(parts, co_p, d_p, ct, ws, cfT):
    specs = _dense_specs(True) + [pl.BlockSpec((1, _LB), lambda i: (0, i))]
    return pl.pallas_call(
        _tc_final_body,
        grid=(_ND,),
        in_specs=specs,
        out_specs=pl.BlockSpec((1, 1), lambda i: (0, 0)),
        out_shape=jax.ShapeDtypeStruct((1, 1), _F32),
    )(parts, co_p, d_p, ct, *ws, cfT)


def _prep_mlp(params):
    (w1, b1), (w2, b2), (w3, b3) = params
    return (w1.T, b1.reshape(-1, 1), w2.T, b2.reshape(-1, 1),
            w3.T, b3.reshape(-1, 1))


def kernel(cart, neighlist, shifts, center_factor, neigh_factor, species,
           params):
    nl = neighlist.astype(_I32)
    ic = nl[0]
    inn = nl[1]
    cartT = jnp.pad(cart.T, ((0, 0), (0, _NP - _N)))
    spT = jnp.pad(species.T, ((0, 0), (0, _NP - _N)))
    cfT = jnp.pad(center_factor.reshape(1, -1), ((0, 0), (0, _NP - _N)))
    shR = shifts.T.reshape(3, _NCH, _C).transpose(1, 0, 2)
    nfR = neigh_factor.reshape(1, _NCH, _C).transpose(1, 0, 2)
    nlR = nl.reshape(2, _NCH, _C).transpose(1, 0, 2)
    ct = jnp.transpose(params["contracted_coeff"][_INDEX_L], (0, 2, 1))
    emb_ws = _prep_mlp(params["embnn"])
    iter_ws = [_prep_mlp(p) for p in params["iters"]]
    out_ws = _prep_mlp(params["outnn"])

    coeff = _tc_embnn(spT, *emb_ws)
    table = jnp.concatenate(
        [cartT, coeff, jnp.zeros((5, _NP), _F32)], axis=0).reshape(1, -1)
    ge = _sc_edge_gather(table, ic, inn)
    P = _tc_edge_math(ge, shR, nfR, nlR)
    parts = _sc_scatter0(P)
    co, dens, qv = _tc_dense_a(parts, ct, iter_ws[0])
    for it in range(1, 3):
        parts = _sc_scatter_iter(P, qv, co)
        co, dens, qv = _tc_dense_b(parts, co, dens, ct, iter_ws[it])
    parts = _sc_scatter_iter(P, qv, co)
    total = _tc_final(parts, co, dens, ct, out_ws, cfT)
    return total[0, 0]
